# Initial kernel scaffold; baseline (speedup 1.0000x reference)
import jax
import jax.numpy as jnp
from jax.experimental import pallas as pl


def kernel(x, x_true):
    raise NotImplementedError("write your pallas kernel here")


# speedup vs baseline: 23.5514x; 23.5514x over previous
import functools

import numpy as np
import jax
import jax.numpy as jnp
from jax import lax
from jax.experimental import pallas as pl
from jax.experimental.pallas import tpu as pltpu
from jax.experimental.pallas import tpu_sc as plsc

_S = 512
_B = 16
_BROWS = 24


def _radius_t() -> np.ndarray:
    col = np.arange(_S, dtype=np.float32)[None, :]
    row = np.arange(_S, dtype=np.float32)[:, None]
    px = col - 256.0
    py = row - 256.0
    r = np.sqrt(px * px + py * py).astype(np.float32)
    r[256, 256] = 1.0
    return r.reshape(_S, 32, 16).transpose(1, 0, 2).copy()


_R_NP = _radius_t()

_mesh = plsc.VectorSubcoreMesh(core_axis_name="c", subcore_axis_name="s")

_GDN = lax.GatherDimensionNumbers(
    offset_dims=(), collapsed_slice_dims=(0,), start_index_map=(0,))


@functools.partial(
    pl.kernel,
    mesh=_mesh,
    out_type=(
        jax.ShapeDtypeStruct((_B, _S, _S), jnp.float32),
        jax.ShapeDtypeStruct((256, 16), jnp.float32),
    ),
    scratch_types=[
        pltpu.VMEM((2, _S, 16), jnp.float32),
        pltpu.VMEM((2, _S, 16), jnp.float32),
        pltpu.VMEM((2, _S, 16), jnp.float32),
        pltpu.VMEM((_BROWS, _S), jnp.float32),
        pltpu.VMEM((8, 16), jnp.float32),
    ],
    compiler_params=pltpu.CompilerParams(
        needs_layout_passes=False, use_tc_tiling_on_sc=False),
)
def _scatter_kernel(x4, xt4, r4, out_hbm, maxes_hbm, xs, xts, rs, band, maxbuf):
    wid = lax.axis_index("s") * 2 + lax.axis_index("c")
    g0 = jnp.minimum(wid, 30)

    rb = jnp.where(wid == 0, 0, 16 * wid + 8)
    band_base = rb * _S
    nrows = jnp.where(wid == 0, 24, jnp.where(wid == 31, 8, 16))
    nwords = nrows * _S

    lane = lax.iota(jnp.int32, 16)
    lane_f = lane.astype(jnp.float32)
    lane15 = lane == 15
    perm = jnp.minimum(lane + 1, 15)[:, None]
    colbase_f = (16 * g0).astype(jnp.float32)
    zeros16 = jnp.zeros((16,), jnp.float32)

    def _zero(rr, c):
        for cc in range(_S // 16):
            band[rr, pl.ds(cc * 16, 16)] = zeros16
        return c

    lax.fori_loop(0, _BROWS, _zero, 0)

    pltpu.sync_copy(r4.at[pl.ds(g0, 2), :, :], rs)

    mvec = zeros16

    for b in range(_B):
        pltpu.sync_copy(x4.at[b, pl.ds(g0, 2), :, :], xs)
        pltpu.sync_copy(xt4.at[b, pl.ds(g0, 2), :, :], xts)

        def _row(rr, c):
            pyf = rr.astype(jnp.float32) - 256.0
            pyv = jnp.full((16,), pyf)
            for v in range(2):
                xv = xs[v, rr, :]
                rv = rs[v, rr, :]
                vals = xts[v, rr, :]
                pxv = (colbase_f + (16.0 * v)) + lane_f - 256.0
                e = jnp.exp(-xv)
                s = 1.0 / (1.0 + e)
                k = 0.4 * s + 0.8
                tx = (k * pxv) / rv
                orow = ((pxv - tx) + 256.0 + 0.5).astype(jnp.int32)
                ty = (k * pyv) / rv
                ocol = ((pyv - ty) + 256.0 + 0.5).astype(jnp.int32)
                local = orow * _S + ocol - band_base
                local = jnp.minimum(jnp.maximum(local, -1), nwords)
                key = local * 16 + lane
                ks, vs = plsc.sort_key_val(key, vals)
                locs = lax.shift_right_arithmetic(ks, 4)
                nxt = lax.gather(locs, perm, _GDN, (1,),
                                 mode=lax.GatherScatterMode.PROMISE_IN_BOUNDS)
                ok = ((locs != nxt) | lane15) & (locs >= 0) & (locs < nwords)
                lrow = lax.shift_right_arithmetic(locs, 9)
                lcol = lax.bitwise_and(locs, 511)
                plsc.store_scatter(band, [lrow, lcol], vs, mask=ok)
            return c

        lax.fori_loop(0, _S, _row, 0)

        @pl.when(wid == 0)
        def _():
            pltpu.sync_copy(band.at[pl.ds(0, 24), :],
                            out_hbm.at[b, pl.ds(0, 24), :])

        @pl.when((wid > 0) & (wid < 31))
        def _():
            pltpu.sync_copy(band.at[pl.ds(0, 16), :],
                            out_hbm.at[b, pl.ds(rb, 16), :])

        @pl.when(wid == 31)
        def _():
            pltpu.sync_copy(band.at[pl.ds(0, 8), :],
                            out_hbm.at[b, pl.ds(504, 8), :])

        def _drain(rr, m):
            for cc in range(_S // 16):
                seg = band[rr, pl.ds(cc * 16, 16)]
                band[rr, pl.ds(cc * 16, 16)] = zeros16
                m = jnp.maximum(m, seg)
            return m

        mvec = lax.fori_loop(0, nrows, _drain, mvec)

    for i in range(8):
        maxbuf[i, :] = mvec
    pltpu.sync_copy(maxbuf, maxes_hbm.at[pl.ds(wid * 8, 8), :])


def _norm_body(o_ref, mx_ref, out_ref):
    m = jnp.max(mx_ref[...])
    out_ref[...] = o_ref[...] / (m + 1e-9)


def _normalize(out3, maxes):
    return pl.pallas_call(
        _norm_body,
        grid=(_B,),
        in_specs=[
            pl.BlockSpec((1, _S, _S), lambda i: (i, 0, 0)),
            pl.BlockSpec((256, 16), lambda i: (0, 0)),
        ],
        out_specs=pl.BlockSpec((1, _S, _S), lambda i: (i, 0, 0)),
        out_shape=jax.ShapeDtypeStruct((_B, _S, _S), jnp.float32),
    )(out3, maxes)


def kernel(x, x_true):
    x4 = jnp.transpose(x.reshape(_B, _S, 32, 16), (0, 2, 1, 3))
    xt4 = jnp.transpose(x_true.reshape(_B, _S, 32, 16), (0, 2, 1, 3))
    r4 = jnp.asarray(_R_NP)
    out3, maxes = _scatter_kernel(x4, xt4, r4)
    out = _normalize(out3, maxes)
    return out.reshape(_B, 1, _S, _S)
